# baseline (speedup 1.0000x reference)
import jax
import jax.numpy as jnp
from jax.experimental import pallas as pl
from jax.experimental.pallas import tpu as pltpu

DIM = 128
K = 65536
B = 1024
RB = 32
CB = K // 2


def _body(s_ref, keys_ref, labels_ref, q_ref, ql_ref,
          outq_ref, outl_ref, outp_ref):
    r = pl.program_id(0)
    c = pl.program_id(1)
    start = pl.multiple_of(s_ref[0], B)
    outq_ref[...] = q_ref[...]

    c0 = c * CB
    @pl.when(jnp.logical_and(start >= c0, start < c0 + CB))
    def _():
        local = pl.multiple_of(start - c0, B)
        outq_ref[:, pl.ds(local, B)] = keys_ref[...]

    @pl.when(jnp.logical_and(r == 0, c == 0))
    def _():
        outl_ref[...] = ql_ref[...]
        outl_ref[:, pl.ds(start, B)] = labels_ref[...]
        outp_ref[0] = s_ref[1]


def kernel(keys, labels, queue, q_label, queue_ptr):
    ptr = queue_ptr[0]
    start = jnp.clip(ptr, 0, K - B)
    new_ptr = (ptr + B) % K
    scalars = jnp.stack([start, new_ptr]).astype(jnp.int32)
    keys_t = keys.T
    labels_row = labels[None, :]

    grid_spec = pltpu.PrefetchScalarGridSpec(
        num_scalar_prefetch=1,
        grid=(DIM // RB, K // CB),
        in_specs=[
            pl.BlockSpec((RB, B), lambda r, c, s: (r, 0)),
            pl.BlockSpec((1, B), lambda r, c, s: (0, 0)),
            pl.BlockSpec((RB, CB), lambda r, c, s: (r, c)),
            pl.BlockSpec((1, K), lambda r, c, s: (0, 0)),
        ],
        out_specs=[
            pl.BlockSpec((RB, CB), lambda r, c, s: (r, c)),
            pl.BlockSpec((1, K), lambda r, c, s: (0, 0)),
            pl.BlockSpec(memory_space=pltpu.SMEM),
        ],
    )
    new_queue, new_q_label, new_queue_ptr = pl.pallas_call(
        _body,
        grid_spec=grid_spec,
        out_shape=[
            jax.ShapeDtypeStruct((DIM, K), jnp.float32),
            jax.ShapeDtypeStruct((1, K), jnp.int32),
            jax.ShapeDtypeStruct((1,), jnp.int32),
        ],
    )(scalars, keys_t, labels_row, queue, q_label)
    return new_queue, new_q_label, new_queue_ptr
